# baseline (speedup 1.0000x reference)
import functools

import jax
import jax.numpy as jnp
from jax.experimental import pallas as pl

N = 50000
E = 800000
G = 16
H = 2
HID = 64
DEPTH = 4


def _mm_kernel(x_ref, w_ref, b_ref, o_ref, *, act):
    y = jnp.dot(x_ref[...], w_ref[...], preferred_element_type=jnp.float32)
    y = y + b_ref[...]
    if act == "relu":
        y = jnp.maximum(y, 0.0)
    o_ref[...] = y


def _linear(x, W, b, act=None, BM=2000):
    M, K = x.shape
    F = W.shape[1]
    b2 = b.reshape(1, F)
    return pl.pallas_call(
        functools.partial(_mm_kernel, act=act),
        grid=(M // BM,),
        in_specs=[
            pl.BlockSpec((BM, K), lambda i: (i, 0)),
            pl.BlockSpec((K, F), lambda i: (0, 0)),
            pl.BlockSpec((1, F), lambda i: (0, 0)),
        ],
        out_specs=pl.BlockSpec((BM, F), lambda i: (i, 0)),
        out_shape=jax.ShapeDtypeStruct((M, F), jnp.float32),
    )(x, W, b2)


def _mm2_kernel(x_ref, w0_ref, b0_ref, w1_ref, b1_ref, o_ref):
    y = jnp.dot(x_ref[...], w0_ref[...], preferred_element_type=jnp.float32)
    y = jnp.maximum(y + b0_ref[...], 0.0)
    z = jnp.dot(y, w1_ref[...], preferred_element_type=jnp.float32)
    o_ref[...] = z + b1_ref[...]


def _mlp2(x, W0, b0, W1, b1, BM=2000):
    M, K = x.shape
    F0 = W0.shape[1]
    F1 = W1.shape[1]
    return pl.pallas_call(
        _mm2_kernel,
        grid=(M // BM,),
        in_specs=[
            pl.BlockSpec((BM, K), lambda i: (i, 0)),
            pl.BlockSpec((K, F0), lambda i: (0, 0)),
            pl.BlockSpec((1, F0), lambda i: (0, 0)),
            pl.BlockSpec((F0, F1), lambda i: (0, 0)),
            pl.BlockSpec((1, F1), lambda i: (0, 0)),
        ],
        out_specs=pl.BlockSpec((BM, F1), lambda i: (i, 0)),
        out_shape=jax.ShapeDtypeStruct((M, F1), jnp.float32),
    )(x, W0, b0.reshape(1, F0), W1, b1.reshape(1, F1))


def _gat_proj_kernel(h_ref, w_ref, a_ref, g_ref, a4_ref):
    g = jnp.dot(h_ref[...], w_ref[...], preferred_element_type=jnp.float32)
    g_ref[...] = g
    a4_ref[...] = jnp.dot(g, a_ref[...], preferred_element_type=jnp.float32)


def _gat_proj(h, W, att_src, att_dst, BM=2000):
    M = h.shape[0]
    K = h.shape[1]
    F = W.shape[1]
    z = jnp.zeros((HID,), jnp.float32)
    A = jnp.stack([
        jnp.concatenate([att_src[0], z]),
        jnp.concatenate([z, att_src[1]]),
        jnp.concatenate([att_dst[0], z]),
        jnp.concatenate([z, att_dst[1]]),
    ], axis=1)
    return pl.pallas_call(
        _gat_proj_kernel,
        grid=(M // BM,),
        in_specs=[
            pl.BlockSpec((BM, K), lambda i: (i, 0)),
            pl.BlockSpec((K, F), lambda i: (0, 0)),
            pl.BlockSpec((F, 4), lambda i: (0, 0)),
        ],
        out_specs=[
            pl.BlockSpec((BM, F), lambda i: (i, 0)),
            pl.BlockSpec((BM, 4), lambda i: (i, 0)),
        ],
        out_shape=[
            jax.ShapeDtypeStruct((M, F), jnp.float32),
            jax.ShapeDtypeStruct((M, 4), jnp.float32),
        ],
    )(h, W, A)


def _out_kernel(fs_ref, b_ref, glw_ref, w0_ref, b0_ref, w1_ref, b1_ref, o_ref):
    bid = b_ref[...]
    onehot = (bid == jax.lax.broadcasted_iota(jnp.int32, (1, G), 1)).astype(jnp.float32)
    y = jnp.dot(fs_ref[...], w0_ref[...], preferred_element_type=jnp.float32)
    y = y + jnp.dot(onehot, glw_ref[...], preferred_element_type=jnp.float32)
    y = jnp.maximum(y + b0_ref[...], 0.0)
    z = jnp.dot(y, w1_ref[...], preferred_element_type=jnp.float32) + b1_ref[...]
    lane = jax.lax.broadcasted_iota(jnp.int32, z.shape, 1)
    z = jnp.where(lane < 24, z, -jnp.inf)
    z = z - jnp.max(z, axis=-1, keepdims=True)
    ez = jnp.exp(z)
    o_ref[...] = ez / jnp.sum(ez, axis=-1, keepdims=True)


def _out_head(fs, batch, gl, p_out0, p_out1, BM=2000):
    M = fs.shape[0]
    W0 = p_out0["W"]
    W0a = W0[:64]
    W0b = W0[64:]
    glw = gl @ W0b
    W1 = p_out1["W"]
    W1p = jnp.pad(W1, ((0, 0), (0, 8)))
    b1p = jnp.pad(p_out1["b"], (0, 8))
    out = pl.pallas_call(
        _out_kernel,
        grid=(M // BM,),
        in_specs=[
            pl.BlockSpec((BM, 64), lambda i: (i, 0)),
            pl.BlockSpec((BM, 1), lambda i: (i, 0)),
            pl.BlockSpec((G, 128), lambda i: (0, 0)),
            pl.BlockSpec((64, 128), lambda i: (0, 0)),
            pl.BlockSpec((1, 128), lambda i: (0, 0)),
            pl.BlockSpec((128, 32), lambda i: (0, 0)),
            pl.BlockSpec((1, 32), lambda i: (0, 0)),
        ],
        out_specs=pl.BlockSpec((BM, 32), lambda i: (i, 0)),
        out_shape=jax.ShapeDtypeStruct((M, 32), jnp.float32),
    )(fs, batch.reshape(M, 1), glw, W0a, p_out0["b"].reshape(1, 128), W1p,
      b1p.reshape(1, 32))
    return out[:, :24]


def _gat_edge(g, a4, src, dst):
    n = g.shape[0]
    a_src = a4[:, :2]
    a_dst = a4[:, 2:]
    alpha = jax.nn.leaky_relu(a_src[src] + a_dst[dst], 0.2)
    amax = jax.ops.segment_max(alpha, dst, num_segments=n)
    ex = jnp.exp(alpha - amax[dst])
    den = jax.ops.segment_sum(ex, dst, num_segments=n)
    coef = ex / (den[dst] + 1e-16)
    g3 = g.reshape(n, H, HID)
    out = jax.ops.segment_sum(g3[src] * coef[:, :, None], dst, num_segments=n)
    return out.mean(axis=1)


def kernel(x, params, edge_index, batch):
    loop = jnp.arange(N, dtype=edge_index.dtype)
    src = jnp.concatenate([edge_index[0], loop])
    dst = jnp.concatenate([edge_index[1], loop])

    fs = _mlp2(x, params["in0"]["W"], params["in0"]["b"],
               params["in1"]["W"], params["in1"]["b"])

    mid = None
    for i in range(DEPTH):
        p0 = params["gat%d_0" % i]
        g, a4 = _gat_proj(fs, p0["W"], p0["att_src"], p0["att_dst"])
        h = jnp.maximum(_gat_edge(g, a4, src, dst) + p0["b"], 0.0)
        p1 = params["gat%d_1" % i]
        g, a4 = _gat_proj(h, p1["W"], p1["att_src"], p1["att_dst"])
        h = _gat_edge(g, a4, src, dst) + p1["b"]
        fs = jnp.maximum(h + fs, 0.0)
        if i == 0:
            mid = fs

    m = _mlp2(mid, params["mid0"]["W"], params["mid0"]["b"],
              params["mid1"]["W"], params["mid1"]["b"])
    gl = jax.ops.segment_max(m, batch, num_segments=G)
    return _out_head(fs, batch, gl, params["out0"], params["out1"])
